# baseline (speedup 1.0000x reference)
import jax
import jax.numpy as jnp
from jax import lax
from jax.experimental import pallas as pl
from jax.experimental.pallas import tpu as pltpu
from jax.experimental.pallas import tpu_sc as plsc

_B, _C, _H, _W = 32, 96, 224, 224
_CCHUNK = 48
_NC = _C // _CCHUNK
_TOPKS = (3, 4, 5, 6, 7, 10, 15, 20)
_NV = 7
_PAD = -1e38


def _tc_stream_prep(in_ref, w1_ref, w2_ref, x_sun_ref, votes_ref, copy_ref,
                    acc_ref):
    b = pl.program_id(0)
    x = in_ref[...]
    copy_ref[...] = x
    rows = jnp.sum(x, axis=2)[0]
    acc_ref[b, 0:40, :] = rows[0:40, :]

    @pl.when(b == _B - 1)
    def _():
        p = acc_ref[:, :33, :]
        sums = jnp.sum(p, axis=2)
        x_sun = sums * (1.0 / (_H * _W))
        x_sun_ref[...] = x_sun

        xt = jnp.tanh(x_sun)
        xg = jax.lax.dot_general(
            xt, w1_ref[...], (((1,), (1,)), ((), ())),
            preferred_element_type=jnp.float32)
        xg = jnp.maximum(xg, 0.0)

        votes = xg[:, None, :] * w2_ref[...][None, :, :]
        votes_ref[:, :, 0:100] = votes
        votes_ref[:, :, 100:112] = jnp.full((_B, 10, 12), _PAD, jnp.float32)


def _sc_topk_kernel(votes_hbm, out_hbm, votes_v, out_v):
    wid = lax.axis_index("s") * 2 + lax.axis_index("c")
    pltpu.sync_copy(votes_hbm.at[wid], votes_v)

    lane = lax.iota(jnp.int32, 16)
    kslot = {k: i for i, k in enumerate(_TOPKS)}
    shuf = [(lane ^ s).astype(jnp.int32) for s in (1, 2, 4, 8)]

    def _bcast_max(v):
        for idx in shuf:
            v = jnp.maximum(v, v.at[idx].get(mode="promise_in_bounds"))
        return v

    def _bcast_sum(v):
        for idx in shuf:
            v = v + v.at[idx].get(mode="promise_in_bounds")
        return v

    def row_body(r, carry):
        vs = [votes_v[r, pl.ds(16 * j, 16)] for j in range(_NV)]
        s = vs[0] + vs[1] + vs[2] + vs[3] + vs[4] + vs[5]
        s = s + jnp.where(lane < 4, vs[6], jnp.float32(0.0))
        linear = _bcast_sum(s)
        rem = list(vs)
        cnt = jnp.zeros((16,), jnp.float32)
        acc = jnp.zeros((16,), jnp.float32)
        recs = {k: jnp.zeros((16,), jnp.float32) for k in _TOPKS}
        for _ in range(max(_TOPKS)):
            mv = rem[0]
            for j in range(1, _NV):
                mv = jnp.maximum(mv, rem[j])
            m = _bcast_max(mv)
            eqs = [rem[j] == m for j in range(_NV)]
            ind = jnp.where(eqs[0], jnp.float32(1.0), jnp.float32(0.0))
            for e in eqs[1:]:
                ind = ind + jnp.where(e, jnp.float32(1.0), jnp.float32(0.0))
            c = _bcast_sum(ind)
            newcnt = cnt + c
            for k in _TOPKS:
                kf = jnp.float32(k)
                val = acc + m * (kf - cnt)
                a = jnp.where(cnt < kf, jnp.float32(1.0), jnp.float32(0.0))
                b = jnp.where(newcnt < kf, jnp.float32(0.0), jnp.float32(1.0))
                hf = a * b
                recs[k] = recs[k] + hf * (val - recs[k])
            acc = acc + m * c
            cnt = newcnt
            rem = [jnp.where(eqs[j], jnp.float32(-3e38), rem[j])
                   for j in range(_NV)]
        out = jnp.zeros((16,), jnp.float32)
        for slot, k in enumerate(_TOPKS):
            out = jnp.where(lane == slot, recs[k], out)
        out = jnp.where(lane == len(_TOPKS), linear, out)
        out_v[r, :] = out
        return carry

    lax.fori_loop(0, 10, row_body, 0)
    pltpu.sync_copy(out_v, out_hbm.at[wid])


def _tc_copy_rest(prev_ref, in_ref, out_ref):
    del prev_ref
    out_ref[...] = in_ref[...]


def kernel(maps, W1, W2):
    x_sun, votes_p, copy_half = pl.pallas_call(
        _tc_stream_prep,
        grid=(_B,),
        in_specs=[pl.BlockSpec((1, _CCHUNK, _H, _W),
                               lambda b: (b, 0, 0, 0)),
                  pl.BlockSpec(W1.shape, lambda b: (0, 0)),
                  pl.BlockSpec(W2.shape, lambda b: (0, 0))],
        out_specs=[pl.BlockSpec((_B, 33), lambda b: (0, 0)),
                   pl.BlockSpec((_B, 10, 16 * _NV), lambda b: (0, 0, 0)),
                   pl.BlockSpec((1, _CCHUNK, _H, _W),
                                lambda b: (b, 0, 0, 0))],
        out_shape=[jax.ShapeDtypeStruct((_B, 33), jnp.float32),
                   jax.ShapeDtypeStruct((_B, 10, 16 * _NV), jnp.float32),
                   jax.ShapeDtypeStruct((_B, _C, _H, _W), jnp.float32)],
        scratch_shapes=[pltpu.VMEM((_B, 40, _W), jnp.float32)],
        compiler_params=pltpu.CompilerParams(
            dimension_semantics=("arbitrary",)),
    )(maps, W1, W2)

    raw = pl.kernel(
        _sc_topk_kernel,
        mesh=plsc.VectorSubcoreMesh(core_axis_name="c", subcore_axis_name="s"),
        out_type=jax.ShapeDtypeStruct((_B, 10, 16), jnp.float32),
        scratch_types=[pltpu.VMEM((10, 16 * _NV), jnp.float32),
                       pltpu.VMEM((10, 16), jnp.float32)],
    )(votes_p)

    maps_copy = pl.pallas_call(
        _tc_copy_rest,
        grid=(_B,),
        in_specs=[pl.BlockSpec(memory_space=pl.ANY),
                  pl.BlockSpec((1, _CCHUNK, _H, _W),
                               lambda b: (b, 1, 0, 0))],
        out_specs=pl.BlockSpec((1, _CCHUNK, _H, _W),
                               lambda b: (b, 1, 0, 0)),
        out_shape=jax.ShapeDtypeStruct((_B, _C, _H, _W), jnp.float32),
        input_output_aliases={0: 0},
        compiler_params=pltpu.CompilerParams(
            dimension_semantics=("arbitrary",)),
    )(copy_half, maps)

    x_son = jnp.transpose(raw[:, :, :9], (2, 0, 1))
    return (x_sun, x_son, maps_copy)


# speedup vs baseline: 1.0005x; 1.0005x over previous
import jax
import jax.numpy as jnp
from jax import lax
from jax.experimental import pallas as pl
from jax.experimental.pallas import tpu as pltpu
from jax.experimental.pallas import tpu_sc as plsc

_B, _C, _H, _W = 32, 96, 224, 224
_CCHUNK = 48
_NC = _C // _CCHUNK
_TOPKS = (3, 4, 5, 6, 7, 10, 15, 20)
_NV = 7
_PAD = -1e38


def _tc_stream_prep(in_ref, w1_ref, w2_ref, x_sun_ref, votes_ref, copy_ref,
                    acc_ref):
    b = pl.program_id(0)
    x = in_ref[...]
    copy_ref[...] = x
    rows = jnp.sum(x, axis=2)[0]
    acc_ref[b, 0:40, :] = rows[0:40, :]

    @pl.when(b == _B - 1)
    def _():
        p = acc_ref[:, :33, :]
        sums = jnp.sum(p, axis=2)
        x_sun = sums * (1.0 / (_H * _W))
        x_sun_ref[...] = x_sun

        xt = jnp.tanh(x_sun)
        xg = jax.lax.dot_general(
            xt, w1_ref[...], (((1,), (1,)), ((), ())),
            preferred_element_type=jnp.float32)
        xg = jnp.maximum(xg, 0.0)

        votes = xg[:, None, :] * w2_ref[...][None, :, :]
        votes_ref[:, :, 0:100] = votes
        votes_ref[:, :, 100:112] = jnp.full((_B, 10, 12), _PAD, jnp.float32)


def _sc_topk_kernel(votes_hbm, out_hbm, votes_v, out_v):
    wid = lax.axis_index("s") * 2 + lax.axis_index("c")
    pltpu.sync_copy(votes_hbm.at[wid], votes_v)

    lane = lax.iota(jnp.int32, 16)
    shuf = [(lane ^ s).astype(jnp.int32) for s in (1, 2, 4, 8)]

    def _bcast_max(v):
        for idx in shuf:
            v = jnp.maximum(v, v.at[idx].get(mode="promise_in_bounds"))
        return v

    def _bcast_sum(v):
        for idx in shuf:
            v = v + v.at[idx].get(mode="promise_in_bounds")
        return v

    def row_body(r, carry):
        vs = [votes_v[r, pl.ds(16 * j, 16)] for j in range(_NV)]
        s = vs[0] + vs[1] + vs[2] + vs[3] + vs[4] + vs[5]
        s = s + jnp.where(lane < 4, vs[6], jnp.float32(0.0))
        linear = _bcast_sum(s)
        rem = list(vs)
        cnt = jnp.zeros((16,), jnp.float32)
        acc = jnp.zeros((16,), jnp.float32)
        recs = {k: jnp.zeros((16,), jnp.float32) for k in _TOPKS}
        for _ in range(max(_TOPKS)):
            mv = rem[0]
            for j in range(1, _NV):
                mv = jnp.maximum(mv, rem[j])
            m = _bcast_max(mv)
            eqs = [rem[j] == m for j in range(_NV)]
            ind = jnp.where(eqs[0], jnp.float32(1.0), jnp.float32(0.0))
            for e in eqs[1:]:
                ind = ind + jnp.where(e, jnp.float32(1.0), jnp.float32(0.0))
            c = _bcast_sum(ind)
            newcnt = cnt + c
            for k in _TOPKS:
                kf = jnp.float32(k)
                val = acc + m * (kf - cnt)
                a = jnp.where(cnt < kf, jnp.float32(1.0), jnp.float32(0.0))
                b = jnp.where(newcnt < kf, jnp.float32(0.0), jnp.float32(1.0))
                hf = a * b
                recs[k] = recs[k] + hf * (val - recs[k])
            acc = acc + m * c
            cnt = newcnt
            rem = [jnp.where(eqs[j], jnp.float32(-3e38), rem[j])
                   for j in range(_NV)]
        out = jnp.zeros((16,), jnp.float32)
        for slot, k in enumerate(_TOPKS):
            out = jnp.where(lane == slot, recs[k], out)
        out = jnp.where(lane == len(_TOPKS), linear, out)
        out_v[r, :] = out
        return carry

    lax.fori_loop(0, 10, row_body, 0)
    pltpu.sync_copy(out_v, out_hbm.at[wid])


def _tc_copy_rest(prev_ref, in_ref, out_ref):
    del prev_ref
    out_ref[...] = in_ref[...]


def kernel(maps, W1, W2):
    x_sun, votes_p, copy_half = pl.pallas_call(
        _tc_stream_prep,
        grid=(_B,),
        in_specs=[pl.BlockSpec((1, _CCHUNK, _H, _W),
                               lambda b: (b, 0, 0, 0)),
                  pl.BlockSpec(W1.shape, lambda b: (0, 0)),
                  pl.BlockSpec(W2.shape, lambda b: (0, 0))],
        out_specs=[pl.BlockSpec((_B, 33), lambda b: (0, 0)),
                   pl.BlockSpec((_B, 10, 16 * _NV), lambda b: (0, 0, 0)),
                   pl.BlockSpec((1, _CCHUNK, _H, _W),
                                lambda b: (b, 0, 0, 0))],
        out_shape=[jax.ShapeDtypeStruct((_B, 33), jnp.float32),
                   jax.ShapeDtypeStruct((_B, 10, 16 * _NV), jnp.float32),
                   jax.ShapeDtypeStruct((_B, _C, _H, _W), jnp.float32)],
        scratch_shapes=[pltpu.VMEM((_B, 40, _W), jnp.float32)],
        compiler_params=pltpu.CompilerParams(
            dimension_semantics=("arbitrary",)),
    )(maps, W1, W2)

    raw = pl.kernel(
        _sc_topk_kernel,
        mesh=plsc.VectorSubcoreMesh(core_axis_name="c", subcore_axis_name="s"),
        out_type=jax.ShapeDtypeStruct((_B, 10, 16), jnp.float32),
        scratch_types=[pltpu.VMEM((10, 16 * _NV), jnp.float32),
                       pltpu.VMEM((10, 16), jnp.float32)],
    )(votes_p)

    maps_copy = pl.pallas_call(
        _tc_copy_rest,
        grid=(_B,),
        in_specs=[pl.BlockSpec(memory_space=pl.ANY),
                  pl.BlockSpec((1, _CCHUNK, _H, _W),
                               lambda b: (b, 1, 0, 0))],
        out_specs=pl.BlockSpec((1, _CCHUNK, _H, _W),
                               lambda b: (b, 1, 0, 0)),
        out_shape=jax.ShapeDtypeStruct((_B, _C, _H, _W), jnp.float32),
        input_output_aliases={0: 0},
        compiler_params=pltpu.CompilerParams(
            dimension_semantics=("arbitrary",)),
    )(copy_half, maps)

    x_son = jnp.transpose(raw[:, :, :9], (2, 0, 1))
    return (x_sun, x_son, maps_copy)
